# baseline (speedup 1.0000x reference)
import functools

import jax
import jax.numpy as jnp
from jax import lax
from jax.experimental import pallas as pl
from jax.experimental.pallas import tpu as pltpu
from jax.experimental.pallas import tpu_sc as plsc

NUM_CORES = 2
NUM_SUBCORES = 16
NUM_WORKERS = NUM_CORES * NUM_SUBCORES
LANES = 16

IDX_CHUNK = 128


def _gather_sc(mols, table128):
    B = mols.shape[0]
    b_per_w = B // NUM_WORKERS
    n_chunks = b_per_w // IDX_CHUNK

    idx3 = mols.reshape(NUM_WORKERS, n_chunks, IDX_CHUNK)

    mesh = plsc.VectorSubcoreMesh(core_axis_name="c", subcore_axis_name="s")

    @functools.partial(
        pl.kernel,
        mesh=mesh,
        out_type=jax.ShapeDtypeStruct((B, 128), jnp.float32),
        scratch_types=[
            pltpu.VMEM((n_chunks, IDX_CHUNK), jnp.int32),
            pltpu.VMEM((n_chunks, IDX_CHUNK), jnp.int32),
            pltpu.VMEM((b_per_w, 128), jnp.float32),
            pltpu.SemaphoreType.DMA,
        ],
        compiler_params=pltpu.CompilerParams(use_tc_tiling_on_sc=False),
    )
    def gather_kernel(idx_hbm, table_hbm, out_hbm, idx_v, row_v, big_v, sem):
        wid = lax.axis_index("s") * NUM_CORES + lax.axis_index("c")
        pltpu.sync_copy(idx_hbm.at[wid], idx_v)
        for c in range(n_chunks):
            for t in range(IDX_CHUNK // LANES):
                sl = pl.ds(t * LANES, LANES)
                row_v[c, sl] = idx_v[c, sl] >> 3
        copies = [
            pltpu.make_async_copy(
                table_hbm.at[row_v.at[c]],
                big_v.at[pl.ds(c * IDX_CHUNK, IDX_CHUNK)],
                sem,
            )
            for c in range(n_chunks)
        ]
        for cp in copies:
            cp.start()
        for cp in copies:
            cp.wait()
        pltpu.sync_copy(big_v, out_hbm.at[pl.ds(wid * b_per_w, b_per_w)])

    return gather_kernel(idx3, table128)


def _scores_tc(packed, onehot, task_table_t):
    B = packed.shape[0]
    D, T = task_table_t.shape
    BB = 1024
    pack = 128 // D

    def mm_kernel(pk_ref, oh_ref, tt_ref, out_ref):
        oh = oh_ref[...]
        ext = pk_ref[:, 0:D] * oh[:, 0:1]
        for g in range(1, pack):
            ext += pk_ref[:, g * D:(g + 1) * D] * oh[:, g:g + 1]
        out_ref[...] = jnp.dot(
            ext, tt_ref[...], preferred_element_type=jnp.float32
        )

    return pl.pallas_call(
        mm_kernel,
        grid=(B // BB,),
        in_specs=[
            pl.BlockSpec((BB, 128), lambda i: (i, 0)),
            pl.BlockSpec((BB, pack), lambda i: (i, 0)),
            pl.BlockSpec((D, T), lambda i: (0, 0)),
        ],
        out_specs=pl.BlockSpec((BB, T), lambda i: (i, 0)),
        out_shape=jax.ShapeDtypeStruct((B, T), jnp.float32),
    )(packed, onehot, task_table_t)


def kernel(mols, mol_table, task_table):
    V, D = mol_table.shape
    pack = 128 // D
    mols = mols.astype(jnp.int32)
    table128 = mol_table.reshape(V // pack, 128)
    packed = _gather_sc(mols, table128)
    onehot = jax.nn.one_hot(mols & (pack - 1), pack, dtype=jnp.float32)
    return _scores_tc(packed, onehot, task_table.T)


# speedup vs baseline: 1.1118x; 1.1118x over previous
import functools

import jax
import jax.numpy as jnp
from jax import lax
from jax.experimental import pallas as pl
from jax.experimental.pallas import tpu as pltpu
from jax.experimental.pallas import tpu_sc as plsc

NUM_CORES = 2
NUM_SUBCORES = 16
NUM_WORKERS = NUM_CORES * NUM_SUBCORES
LANES = 16

IDX_CHUNK = 128


def _gather_sc(mols, table128):
    B = mols.shape[0]
    b_per_w = B // NUM_WORKERS
    n_chunks = b_per_w // IDX_CHUNK

    idx3 = mols.reshape(NUM_WORKERS, n_chunks, IDX_CHUNK)

    mesh = plsc.VectorSubcoreMesh(core_axis_name="c", subcore_axis_name="s")

    @functools.partial(
        pl.kernel,
        mesh=mesh,
        out_type=jax.ShapeDtypeStruct((B, 128), jnp.float32),
        scratch_types=[
            pltpu.VMEM((n_chunks, IDX_CHUNK), jnp.int32),
            pltpu.VMEM((n_chunks, IDX_CHUNK), jnp.int32),
            pltpu.VMEM((b_per_w, 128), jnp.float32),
            pltpu.SemaphoreType.DMA,
        ],
        compiler_params=pltpu.CompilerParams(use_tc_tiling_on_sc=False),
    )
    def gather_kernel(idx_hbm, table_hbm, out_hbm, idx_v, row_v, big_v, sem):
        wid = lax.axis_index("s") * NUM_CORES + lax.axis_index("c")
        pltpu.sync_copy(idx_hbm.at[wid], idx_v)
        for c in range(n_chunks):
            for t in range(IDX_CHUNK // LANES):
                sl = pl.ds(t * LANES, LANES)
                row_v[c, sl] = idx_v[c, sl] >> 3
        copies = [
            pltpu.make_async_copy(
                table_hbm.at[row_v.at[c]],
                big_v.at[pl.ds(c * IDX_CHUNK, IDX_CHUNK)],
                sem,
            )
            for c in range(n_chunks)
        ]
        for cp in copies:
            cp.start()
        for cp in copies:
            cp.wait()
        pltpu.sync_copy(big_v, out_hbm.at[pl.ds(wid * b_per_w, b_per_w)])

    return gather_kernel(idx3, table128)


def _scores_tc(packed, onehot, task_table):
    B = packed.shape[0]
    T, D = task_table.shape
    BB = 1024
    pack = 128 // D

    def mm_kernel(pk_ref, oh_ref, tt_ref, out_ref):
        oh = oh_ref[...]
        ext = pk_ref[:, 0:D] * oh[:, 0:1]
        for g in range(1, pack):
            ext += pk_ref[:, g * D:(g + 1) * D] * oh[:, g:g + 1]
        out_ref[...] = lax.dot_general(
            tt_ref[...], ext,
            (((1,), (1,)), ((), ())),
            preferred_element_type=jnp.float32,
        )

    return pl.pallas_call(
        mm_kernel,
        grid=(B // BB,),
        in_specs=[
            pl.BlockSpec((BB, 128), lambda i: (i, 0)),
            pl.BlockSpec((BB, pack), lambda i: (i, 0)),
            pl.BlockSpec((T, D), lambda i: (0, 0)),
        ],
        out_specs=pl.BlockSpec((T, BB), lambda i: (0, i)),
        out_shape=jax.ShapeDtypeStruct((T, B), jnp.float32),
    )(packed, onehot, task_table)


def kernel(mols, mol_table, task_table):
    V, D = mol_table.shape
    pack = 128 // D
    mols = mols.astype(jnp.int32)
    table128 = mol_table.reshape(V // pack, 128)
    packed = _gather_sc(mols, table128)
    onehot = jax.nn.one_hot(mols & (pack - 1), pack, dtype=jnp.float32)
    return _scores_tc(packed, onehot, task_table).T


# speedup vs baseline: 1.1148x; 1.0027x over previous
import functools

import jax
import jax.numpy as jnp
from jax import lax
from jax.experimental import pallas as pl
from jax.experimental.pallas import tpu as pltpu
from jax.experimental.pallas import tpu_sc as plsc

NUM_CORES = 2
NUM_SUBCORES = 16
NUM_WORKERS = NUM_CORES * NUM_SUBCORES
LANES = 16

IDX_CHUNK = 128


def _gather_sc(mols, table128):
    B = mols.shape[0]
    b_per_w = B // NUM_WORKERS
    n_chunks = b_per_w // IDX_CHUNK

    idx3 = mols.reshape(NUM_WORKERS, n_chunks, IDX_CHUNK)

    mesh = plsc.VectorSubcoreMesh(core_axis_name="c", subcore_axis_name="s")

    @functools.partial(
        pl.kernel,
        mesh=mesh,
        out_type=jax.ShapeDtypeStruct((B, 128), jnp.float32),
        scratch_types=[
            pltpu.VMEM((n_chunks, IDX_CHUNK), jnp.int32),
            pltpu.VMEM((n_chunks, IDX_CHUNK), jnp.int32),
            pltpu.VMEM((b_per_w, 128), jnp.float32),
            pltpu.SemaphoreType.DMA,
        ],
    )
    def gather_kernel(idx_hbm, table_hbm, out_hbm, idx_v, row_v, big_v, sem):
        wid = lax.axis_index("s") * NUM_CORES + lax.axis_index("c")
        pltpu.sync_copy(idx_hbm.at[wid], idx_v)
        for c in range(n_chunks):
            for t in range(IDX_CHUNK // LANES):
                sl = pl.ds(t * LANES, LANES)
                row_v[c, sl] = idx_v[c, sl] >> 3
        copies = [
            pltpu.make_async_copy(
                table_hbm.at[row_v.at[c]],
                big_v.at[pl.ds(c * IDX_CHUNK, IDX_CHUNK)],
                sem,
            )
            for c in range(n_chunks)
        ]
        for cp in copies:
            cp.start()
        for cp in copies:
            cp.wait()
        pltpu.sync_copy(big_v, out_hbm.at[pl.ds(wid * b_per_w, b_per_w)])

    return gather_kernel(idx3, table128)


def _scores_tc(packed, onehot, task_table):
    B = packed.shape[0]
    T, D = task_table.shape
    BB = 1024
    pack = 128 // D

    def mm_kernel(pk_ref, oh_ref, tt_ref, out_ref):
        oh = oh_ref[...]
        ext = pk_ref[:, 0:D] * oh[:, 0:1]
        for g in range(1, pack):
            ext += pk_ref[:, g * D:(g + 1) * D] * oh[:, g:g + 1]
        out_ref[...] = lax.dot_general(
            tt_ref[...], ext,
            (((1,), (1,)), ((), ())),
            preferred_element_type=jnp.float32,
        )

    return pl.pallas_call(
        mm_kernel,
        grid=(B // BB,),
        in_specs=[
            pl.BlockSpec((BB, 128), lambda i: (i, 0)),
            pl.BlockSpec((BB, pack), lambda i: (i, 0)),
            pl.BlockSpec((T, D), lambda i: (0, 0)),
        ],
        out_specs=pl.BlockSpec((T, BB), lambda i: (0, i)),
        out_shape=jax.ShapeDtypeStruct((T, B), jnp.float32),
    )(packed, onehot, task_table)


def kernel(mols, mol_table, task_table):
    V, D = mol_table.shape
    pack = 128 // D
    mols = mols.astype(jnp.int32)
    table128 = mol_table.reshape(V // pack, 128)
    packed = _gather_sc(mols, table128)
    onehot = jax.nn.one_hot(mols & (pack - 1), pack, dtype=jnp.float32)
    return _scores_tc(packed, onehot, task_table).T
